# baseline (speedup 1.0000x reference)
import jax
import jax.numpy as jnp
from jax import lax
from jax.experimental import pallas as pl
from jax.experimental.pallas import tpu as pltpu
from jax.experimental.pallas import tpu_sc as plsc

_N = 16777216
_BINS = 50
_EPS = 1e-08

_NC = 2
_NS = 16
_NW = _NC * _NS
_CH = _N // _NW
_T = 16384
_TILES = _CH // _T
_NBUF = 4
_U = 8
_MMU = 8
_PAD_BINS = 64


def _mesh():
    return plsc.VectorSubcoreMesh(
        core_axis_name="c", subcore_axis_name="s", num_cores=_NC, num_subcores=_NS
    )


def _wid():
    return lax.axis_index("s") * _NC + lax.axis_index("c")


def _mm_body(cv_hbm, iv_hbm, min_c_hbm, max_c_hbm, min_i_hbm, max_i_hbm,
             buf0, buf1, buf2, buf3, st, sem0, sem1, sem2, sem3):
    wid = _wid()
    base = wid * _CH
    inf = jnp.full((16,), jnp.inf, jnp.float32)
    ninf = jnp.full((16,), -jnp.inf, jnp.float32)
    ring = ((buf0, sem0), (buf1, sem1), (buf2, sem2), (buf3, sem3))

    for src, out_min, out_max in ((cv_hbm, min_c_hbm, max_c_hbm),
                                  (iv_hbm, min_i_hbm, max_i_hbm)):
        def dma(t, buf, sem):
            return pltpu.make_async_copy(src.at[pl.ds(base + t * _T, _T)], buf, sem)

        for b, (buf, sem) in enumerate(ring):
            dma(b, buf, sem).start()

        def outer(g, carry):
            for b, (buf, sem) in enumerate(ring):
                t = _NBUF * g + b
                dma(t, buf, sem).wait()

                def inner(i, acc):
                    mns, mxs = acc
                    i0 = i * (16 * _MMU)
                    new_mns, new_mxs = [], []
                    for uu in range(_MMU):
                        v = buf[pl.ds(i0 + uu * 16, 16)]
                        new_mns.append(jnp.minimum(mns[uu], v))
                        new_mxs.append(jnp.maximum(mxs[uu], v))
                    return tuple(new_mns), tuple(new_mxs)

                carry = lax.fori_loop(0, _T // (16 * _MMU), inner, carry)

                @pl.when(t + _NBUF < _TILES)
                def _():
                    dma(t + _NBUF, buf, sem).start()
            return carry

        init = (tuple(inf for _ in range(_MMU)), tuple(ninf for _ in range(_MMU)))
        mns, mxs = lax.fori_loop(0, _TILES // _NBUF, outer, init)
        mn, mx = mns[0], mxs[0]
        for uu in range(1, _MMU):
            mn = jnp.minimum(mn, mns[uu])
            mx = jnp.maximum(mx, mxs[uu])
        st[0] = mn
        pltpu.sync_copy(st.at[0], out_min.at[wid])
        st[1] = mx
        pltpu.sync_copy(st.at[1], out_max.at[wid])


def _sc_minmax(cv, iv):
    part = jax.ShapeDtypeStruct((_NW, 16), jnp.float32)
    return pl.kernel(
        _mm_body,
        out_type=[part] * 4,
        mesh=_mesh(),
        scratch_types=[
            pltpu.VMEM((_T,), jnp.float32),
            pltpu.VMEM((_T,), jnp.float32),
            pltpu.VMEM((_T,), jnp.float32),
            pltpu.VMEM((_T,), jnp.float32),
            pltpu.VMEM((2, 16), jnp.float32),
            pltpu.SemaphoreType.DMA,
            pltpu.SemaphoreType.DMA,
            pltpu.SemaphoreType.DMA,
            pltpu.SemaphoreType.DMA,
        ],
        compiler_params=pltpu.CompilerParams(needs_layout_passes=False),
    )(cv, iv)


def _hist_body(cv_hbm, iv_hbm, min_c_hbm, max_c_hbm, min_i_hbm, max_i_hbm,
               out_c_hbm, out_i_hbm, mm_hbm,
               buf0, buf1, buf2, buf3, h, h2, pm, st4, sem0, sem1, sem2, sem3):
    wid = _wid()
    base = wid * _CH
    ring = ((buf0, sem0), (buf1, sem1), (buf2, sem2), (buf3, sem3))
    lane = lax.iota(jnp.int32, 16)
    ones = jnp.ones((16,), jnp.float32)
    zeros = jnp.zeros((16,), jnp.float32)
    limit = jnp.full((16,), float(_BINS - 1), jnp.float32)

    def global_reduce(part_hbm, combine, init):
        pltpu.sync_copy(part_hbm, pm)
        acc = jnp.full((16,), init, jnp.float32)
        for j in range(_NW):
            acc = combine(acc, pm[j])
        return combine(acc, jnp.full((16,), init, jnp.float32))

    vecs = []
    for part_hbm, combine, init, red in (
        (min_c_hbm, jnp.minimum, jnp.inf, jnp.min),
        (max_c_hbm, jnp.maximum, -jnp.inf, jnp.max),
        (min_i_hbm, jnp.minimum, jnp.inf, jnp.min),
        (max_i_hbm, jnp.maximum, -jnp.inf, jnp.max),
    ):
        vec = global_reduce(part_hbm, combine, init)
        vecs.append(jnp.broadcast_to(red(vec), (16,)))
    cminv, cmaxv, iminv, imaxv = vecs

    for j, v in enumerate(vecs):
        st4[j] = v

    @pl.when(wid == 0)
    def _():
        pltpu.sync_copy(st4, mm_hbm)

    binsv = jnp.full((16,), float(_BINS), jnp.float32)
    epsv = jnp.full((16,), 1e-12, jnp.float32)
    for src, vmin, vmaxv, out in ((cv_hbm, cminv, cmaxv, out_c_hbm),
                                  (iv_hbm, iminv, imaxv, out_i_hbm)):
        denomv = (vmaxv - vmin) / binsv + epsv
        for j in range(_PAD_BINS):
            h[pl.ds(j * 16, 16)] = zeros

        def dma(t, buf, sem):
            return pltpu.make_async_copy(src.at[pl.ds(base + t * _T, _T)], buf, sem)

        for b, (buf, sem) in enumerate(ring):
            dma(b, buf, sem).start()

        def outer(g, carry):
            for b, (buf, sem) in enumerate(ring):
                t = _NBUF * g + b
                dma(t, buf, sem).wait()

                @plsc.parallel_loop(0, _T // 16, unroll=_U)
                def _(i):
                    v = buf[pl.ds(i * 16, 16)]
                    q = (v - vmin) / denomv
                    idx = jnp.minimum(q, limit).astype(jnp.int32)
                    plsc.addupdate_scatter(h, [idx * 16 + lane], ones)

                @pl.when(t + _NBUF < _TILES)
                def _():
                    dma(t + _NBUF, buf, sem).start()
            return carry

        lax.fori_loop(0, _TILES // _NBUF, outer, 0)
        for g in range(_PAD_BINS // 16):
            acc = zeros
            for j in range(16):
                s = jnp.sum(h[pl.ds((g * 16 + j) * 16, 16)])
                acc = jnp.where(lane == j, s, acc)
            h2[pl.ds(g * 16, 16)] = acc
        pltpu.sync_copy(h2, out.at[wid])


def _sc_hist(cv, iv, min_c, max_c, min_i, max_i):
    sub = jax.ShapeDtypeStruct((_NW, _PAD_BINS), jnp.float32)
    mm = jax.ShapeDtypeStruct((4, 16), jnp.float32)
    return pl.kernel(
        _hist_body,
        out_type=[sub, sub, mm],
        mesh=_mesh(),
        scratch_types=[
            pltpu.VMEM((_T,), jnp.float32),
            pltpu.VMEM((_T,), jnp.float32),
            pltpu.VMEM((_T,), jnp.float32),
            pltpu.VMEM((_T,), jnp.float32),
            pltpu.VMEM((16 * _PAD_BINS,), jnp.float32),
            pltpu.VMEM((_PAD_BINS,), jnp.float32),
            pltpu.VMEM((_NW, 16), jnp.float32),
            pltpu.VMEM((4, 16), jnp.float32),
            pltpu.SemaphoreType.DMA,
            pltpu.SemaphoreType.DMA,
            pltpu.SemaphoreType.DMA,
            pltpu.SemaphoreType.DMA,
        ],
        compiler_params=pltpu.CompilerParams(needs_layout_passes=False),
    )(cv, iv, min_c, max_c, min_i, max_i)


def _red_body(a_ref, b_ref, oa_ref, ob_ref):
    oa_ref[...] = jnp.sum(a_ref[...], axis=0, keepdims=True)
    ob_ref[...] = jnp.sum(b_ref[...], axis=0, keepdims=True)


def _reduce(a, b):
    out = jax.ShapeDtypeStruct((1, _PAD_BINS), jnp.float32)
    return pl.pallas_call(_red_body, out_shape=[out, out])(a, b)


def kernel(current_values, initial_values):
    min_c, max_c, min_i, max_i = _sc_minmax(current_values, initial_values)
    sub_c, sub_i, mm = _sc_hist(current_values, initial_values,
                                min_c, max_c, min_i, max_i)
    red_c, red_i = _reduce(sub_c, sub_i)
    counts_c = red_c[0, :_BINS]
    counts_i = red_i[0, :_BINS]
    cmin, cmax, imin, imax = mm[0, 0], mm[1, 0], mm[2, 0], mm[3, 0]

    def hist_of(counts, vmin, vmax):
        width = (vmax - vmin) / _BINS
        density = counts / (counts.sum() * width + 1e-12)
        return density / (density.sum() + _EPS)

    current_hist = hist_of(counts_c, cmin, cmax)
    initial_hist = hist_of(counts_i, imin, imax)
    p = initial_hist + _EPS
    q = current_hist + _EPS
    p = p / p.sum()
    q = q / q.sum()
    m = 0.5 * (p + q)
    kl_m_p = jnp.sum(m * (jnp.log(m) - jnp.log(p)))
    kl_m_q = jnp.sum(m * (jnp.log(m) - jnp.log(q)))
    js_div = 0.5 * kl_m_p + 0.5 * kl_m_q
    avg_js = js_div / 1.0
    reward = -avg_js
    return reward


# speedup vs baseline: 1.0079x; 1.0079x over previous
import jax
import jax.numpy as jnp
from jax import lax
from jax.experimental import pallas as pl
from jax.experimental.pallas import tpu as pltpu
from jax.experimental.pallas import tpu_sc as plsc

_N = 16777216
_BINS = 50
_EPS = 1e-08

_NC = 2
_NS = 16
_NW = _NC * _NS
_CH = _N // _NW
_T = 16384
_TILES = _CH // _T
_NBUF = 4
_U = 16
_MMU = 8
_PAD_BINS = 64


def _mesh():
    return plsc.VectorSubcoreMesh(
        core_axis_name="c", subcore_axis_name="s", num_cores=_NC, num_subcores=_NS
    )


def _wid():
    return lax.axis_index("s") * _NC + lax.axis_index("c")


def _mm_body(cv_hbm, iv_hbm, min_c_hbm, max_c_hbm, min_i_hbm, max_i_hbm,
             buf0, buf1, buf2, buf3, st, sem0, sem1, sem2, sem3):
    wid = _wid()
    base = wid * _CH
    inf = jnp.full((16,), jnp.inf, jnp.float32)
    ninf = jnp.full((16,), -jnp.inf, jnp.float32)
    ring = ((buf0, sem0), (buf1, sem1), (buf2, sem2), (buf3, sem3))

    for src, out_min, out_max in ((cv_hbm, min_c_hbm, max_c_hbm),
                                  (iv_hbm, min_i_hbm, max_i_hbm)):
        def dma(t, buf, sem):
            return pltpu.make_async_copy(src.at[pl.ds(base + t * _T, _T)], buf, sem)

        for b, (buf, sem) in enumerate(ring):
            dma(b, buf, sem).start()

        def outer(g, carry):
            for b, (buf, sem) in enumerate(ring):
                t = _NBUF * g + b
                dma(t, buf, sem).wait()

                def inner(i, acc):
                    mns, mxs = acc
                    i0 = i * (16 * _MMU)
                    new_mns, new_mxs = [], []
                    for uu in range(_MMU):
                        v = buf[pl.ds(i0 + uu * 16, 16)]
                        new_mns.append(jnp.minimum(mns[uu], v))
                        new_mxs.append(jnp.maximum(mxs[uu], v))
                    return tuple(new_mns), tuple(new_mxs)

                carry = lax.fori_loop(0, _T // (16 * _MMU), inner, carry)

                @pl.when(t + _NBUF < _TILES)
                def _():
                    dma(t + _NBUF, buf, sem).start()
            return carry

        init = (tuple(inf for _ in range(_MMU)), tuple(ninf for _ in range(_MMU)))
        mns, mxs = lax.fori_loop(0, _TILES // _NBUF, outer, init)
        mn, mx = mns[0], mxs[0]
        for uu in range(1, _MMU):
            mn = jnp.minimum(mn, mns[uu])
            mx = jnp.maximum(mx, mxs[uu])
        st[0] = mn
        pltpu.sync_copy(st.at[0], out_min.at[wid])
        st[1] = mx
        pltpu.sync_copy(st.at[1], out_max.at[wid])


def _sc_minmax(cv, iv):
    part = jax.ShapeDtypeStruct((_NW, 16), jnp.float32)
    return pl.kernel(
        _mm_body,
        out_type=[part] * 4,
        mesh=_mesh(),
        scratch_types=[
            pltpu.VMEM((_T,), jnp.float32),
            pltpu.VMEM((_T,), jnp.float32),
            pltpu.VMEM((_T,), jnp.float32),
            pltpu.VMEM((_T,), jnp.float32),
            pltpu.VMEM((2, 16), jnp.float32),
            pltpu.SemaphoreType.DMA,
            pltpu.SemaphoreType.DMA,
            pltpu.SemaphoreType.DMA,
            pltpu.SemaphoreType.DMA,
        ],
        compiler_params=pltpu.CompilerParams(needs_layout_passes=False),
    )(cv, iv)


def _hist_body(cv_hbm, iv_hbm, min_c_hbm, max_c_hbm, min_i_hbm, max_i_hbm,
               out_c_hbm, out_i_hbm, mm_hbm,
               buf0, buf1, buf2, buf3, h, h2, pm, st4, sem0, sem1, sem2, sem3):
    wid = _wid()
    base = wid * _CH
    ring = ((buf0, sem0), (buf1, sem1), (buf2, sem2), (buf3, sem3))
    lane = lax.iota(jnp.int32, 16)
    ones = jnp.ones((16,), jnp.float32)
    zeros = jnp.zeros((16,), jnp.float32)
    limit = jnp.full((16,), float(_BINS - 1), jnp.float32)

    def global_reduce(part_hbm, combine, init):
        pltpu.sync_copy(part_hbm, pm)
        acc = jnp.full((16,), init, jnp.float32)
        for j in range(_NW):
            acc = combine(acc, pm[j])
        return combine(acc, jnp.full((16,), init, jnp.float32))

    vecs = []
    for part_hbm, combine, init, red in (
        (min_c_hbm, jnp.minimum, jnp.inf, jnp.min),
        (max_c_hbm, jnp.maximum, -jnp.inf, jnp.max),
        (min_i_hbm, jnp.minimum, jnp.inf, jnp.min),
        (max_i_hbm, jnp.maximum, -jnp.inf, jnp.max),
    ):
        vec = global_reduce(part_hbm, combine, init)
        vecs.append(jnp.broadcast_to(red(vec), (16,)))
    cminv, cmaxv, iminv, imaxv = vecs

    for j, v in enumerate(vecs):
        st4[j] = v

    @pl.when(wid == 0)
    def _():
        pltpu.sync_copy(st4, mm_hbm)

    binsv = jnp.full((16,), float(_BINS), jnp.float32)
    epsv = jnp.full((16,), 1e-12, jnp.float32)
    for src, vmin, vmaxv, out in ((cv_hbm, cminv, cmaxv, out_c_hbm),
                                  (iv_hbm, iminv, imaxv, out_i_hbm)):
        denomv = (vmaxv - vmin) / binsv + epsv
        for j in range(_PAD_BINS):
            h[pl.ds(j * 16, 16)] = zeros

        def dma(t, buf, sem):
            return pltpu.make_async_copy(src.at[pl.ds(base + t * _T, _T)], buf, sem)

        for b, (buf, sem) in enumerate(ring):
            dma(b, buf, sem).start()

        def outer(g, carry):
            for b, (buf, sem) in enumerate(ring):
                t = _NBUF * g + b
                dma(t, buf, sem).wait()

                @plsc.parallel_loop(0, _T // 16, unroll=_U)
                def _(i):
                    v = buf[pl.ds(i * 16, 16)]
                    q = (v - vmin) / denomv
                    idx = jnp.minimum(q, limit).astype(jnp.int32)
                    plsc.addupdate_scatter(h, [idx * 16 + lane], ones)

                @pl.when(t + _NBUF < _TILES)
                def _():
                    dma(t + _NBUF, buf, sem).start()
            return carry

        lax.fori_loop(0, _TILES // _NBUF, outer, 0)
        for g in range(_PAD_BINS // 16):
            acc = zeros
            for j in range(16):
                s = jnp.sum(h[pl.ds((g * 16 + j) * 16, 16)])
                acc = jnp.where(lane == j, s, acc)
            h2[pl.ds(g * 16, 16)] = acc
        pltpu.sync_copy(h2, out.at[wid])


def _sc_hist(cv, iv, min_c, max_c, min_i, max_i):
    sub = jax.ShapeDtypeStruct((_NW, _PAD_BINS), jnp.float32)
    mm = jax.ShapeDtypeStruct((4, 16), jnp.float32)
    return pl.kernel(
        _hist_body,
        out_type=[sub, sub, mm],
        mesh=_mesh(),
        scratch_types=[
            pltpu.VMEM((_T,), jnp.float32),
            pltpu.VMEM((_T,), jnp.float32),
            pltpu.VMEM((_T,), jnp.float32),
            pltpu.VMEM((_T,), jnp.float32),
            pltpu.VMEM((16 * _PAD_BINS,), jnp.float32),
            pltpu.VMEM((_PAD_BINS,), jnp.float32),
            pltpu.VMEM((_NW, 16), jnp.float32),
            pltpu.VMEM((4, 16), jnp.float32),
            pltpu.SemaphoreType.DMA,
            pltpu.SemaphoreType.DMA,
            pltpu.SemaphoreType.DMA,
            pltpu.SemaphoreType.DMA,
        ],
        compiler_params=pltpu.CompilerParams(needs_layout_passes=False),
    )(cv, iv, min_c, max_c, min_i, max_i)


def _red_body(a_ref, b_ref, oa_ref, ob_ref):
    oa_ref[...] = jnp.sum(a_ref[...], axis=0, keepdims=True)
    ob_ref[...] = jnp.sum(b_ref[...], axis=0, keepdims=True)


def _reduce(a, b):
    out = jax.ShapeDtypeStruct((1, _PAD_BINS), jnp.float32)
    return pl.pallas_call(_red_body, out_shape=[out, out])(a, b)


def kernel(current_values, initial_values):
    min_c, max_c, min_i, max_i = _sc_minmax(current_values, initial_values)
    sub_c, sub_i, mm = _sc_hist(current_values, initial_values,
                                min_c, max_c, min_i, max_i)
    red_c, red_i = _reduce(sub_c, sub_i)
    counts_c = red_c[0, :_BINS]
    counts_i = red_i[0, :_BINS]
    cmin, cmax, imin, imax = mm[0, 0], mm[1, 0], mm[2, 0], mm[3, 0]

    def hist_of(counts, vmin, vmax):
        width = (vmax - vmin) / _BINS
        density = counts / (counts.sum() * width + 1e-12)
        return density / (density.sum() + _EPS)

    current_hist = hist_of(counts_c, cmin, cmax)
    initial_hist = hist_of(counts_i, imin, imax)
    p = initial_hist + _EPS
    q = current_hist + _EPS
    p = p / p.sum()
    q = q / q.sum()
    m = 0.5 * (p + q)
    kl_m_p = jnp.sum(m * (jnp.log(m) - jnp.log(p)))
    kl_m_q = jnp.sum(m * (jnp.log(m) - jnp.log(q)))
    js_div = 0.5 * kl_m_p + 0.5 * kl_m_q
    avg_js = js_div / 1.0
    reward = -avg_js
    return reward
